# baseline (speedup 1.0000x reference)
import jax
import jax.numpy as jnp
from jax.experimental import pallas as pl
from jax.experimental.pallas import tpu as pltpu

_B, _C, _T, _F = 16, 2, 2000, 257
_TB = 512
_NT = -(-_T // _TB)
_NSTEP = 9


def _ema_norm_kernel(x_ref, s1_ref, a_ref, p_ref, dec_ref, w_ref, b_ref,
                     res_ref, sm_ref, carry_ref):
    t = pl.program_id(1)

    @pl.when(t == 0)
    def _():
        carry_ref[...] = pltpu.repeat(s1_ref[0], _TB // 128, axis=2)

    x0 = x_ref[0, 0]
    x1 = x_ref[0, 1]
    p0 = x0 * x0
    p1 = x1 * x1
    d2_0 = p0 + jnp.roll(p0, 1, axis=1)
    d2_1 = p1 + jnp.roll(p1, 1, axis=1)

    liota = jax.lax.broadcasted_iota(jnp.int32, (_F, 2, _TB), 2)
    y = jnp.where(liota < _T - t * _TB, d2_0 * a_ref[...], 0.0)
    d = 1
    for k in range(_NSTEP):
        y = y + dec_ref[k] * jnp.roll(y, d, axis=2)
        d *= 2

    s = y + p_ref[...] * carry_ref[...]
    carry_ref[...] = jnp.broadcast_to(s[:, :, _TB - 1:_TB], s.shape)

    smooth = jnp.sqrt(s)
    sm_ref[0] = smooth

    wr = pltpu.repeat(w_ref[...], _TB // 128, axis=3)
    br = pltpu.repeat(b_ref[...], _TB // 128, axis=3)
    res_ref[0, 0] = x0 / (smooth + 1e-8) * wr[0] + br[0]
    res_ref[0, 1] = x1 / (jnp.sqrt(d2_1) + 1e-8) * wr[1] + br[1]


def kernel(input, s_1, weights, bias, alpha_param):
    B, C, T, F, TB = _B, _C, _T, _F, _TB

    x5 = input.transpose(0, 1, 3, 4, 2)

    a = jax.nn.sigmoid(alpha_param.reshape(F))
    la = jnp.log1p(-a)
    liota = jnp.arange(TB, dtype=jnp.float32)
    p_c = jnp.exp(la[:, None] * (liota[None, :] + 1.0))
    decs = []
    d = 1
    for _ in range(_NSTEP):
        dec = jnp.where(liota[None, :] >= d,
                        jnp.exp(la * float(d))[:, None], 0.0)
        decs.append(dec)
        d *= 2
    dec_c = jnp.stack(decs, axis=0)

    dup = lambda v: jnp.broadcast_to(v[..., None, :], v.shape[:-1] + (2, v.shape[-1]))
    a_full = dup(jnp.broadcast_to(a[:, None], (F, TB)))
    p_full = dup(p_c)
    dec_full = dup(dec_c)
    s1_b = jnp.broadcast_to(s_1.reshape(B, F, 1, 1), (B, F, 2, 128))
    w_b = jnp.broadcast_to(weights.reshape(C, F, 1, 1), (C, F, 2, 128))
    b_b = jnp.broadcast_to(bias.reshape(C, F, 1, 1), (C, F, 2, 128))

    res5, smooth5 = pl.pallas_call(
        _ema_norm_kernel,
        grid=(B, _NT),
        in_specs=[
            pl.BlockSpec((1, C, F, 2, TB), lambda b, t: (b, 0, 0, 0, t)),
            pl.BlockSpec((1, F, 2, 128), lambda b, t: (b, 0, 0, 0)),
            pl.BlockSpec((F, 2, TB), lambda b, t: (0, 0, 0)),
            pl.BlockSpec((F, 2, TB), lambda b, t: (0, 0, 0)),
            pl.BlockSpec((_NSTEP, F, 2, TB), lambda b, t: (0, 0, 0, 0)),
            pl.BlockSpec((C, F, 2, 128), lambda b, t: (0, 0, 0, 0)),
            pl.BlockSpec((C, F, 2, 128), lambda b, t: (0, 0, 0, 0)),
        ],
        out_specs=[
            pl.BlockSpec((1, C, F, 2, TB), lambda b, t: (b, 0, 0, 0, t)),
            pl.BlockSpec((1, F, 2, TB), lambda b, t: (b, 0, 0, t)),
        ],
        out_shape=[
            jax.ShapeDtypeStruct((B, C, F, 2, T), jnp.float32),
            jax.ShapeDtypeStruct((B, F, 2, T), jnp.float32),
        ],
        scratch_shapes=[pltpu.VMEM((F, 2, TB), jnp.float32)],
        compiler_params=pltpu.CompilerParams(
            dimension_semantics=("parallel", "arbitrary"),
        ),
        name="ema_norm",
    )(x5, s1_b, a_full, p_full, dec_full, w_b, b_b)

    res = res5.transpose(0, 1, 4, 2, 3)
    sm = smooth5[:, :, 0, :]
    smooth_data = sm.transpose(0, 2, 1).reshape(B, 1, T, F, 1)
    s_final = (sm[:, :, T - 1] ** 2).reshape(B, 1, F, 1)
    return res, s_final, smooth_data


# speedup vs baseline: 1.9750x; 1.9750x over previous
import jax
import jax.numpy as jnp
from jax.experimental import pallas as pl
from jax.experimental.pallas import tpu as pltpu

_B, _C, _T, _F = 16, 2, 2000, 257
_TB = 512
_NT = -(-_T // _TB)
_NSTEP = 9


def _ema_norm_kernel(xr_ref, xi_ref, s1_ref, a_ref, p_ref, dec_ref,
                     w_ref, b_ref, rr_ref, ri_ref, sm_ref, carry_ref):
    t = pl.program_id(1)

    @pl.when(t == 0)
    def _():
        carry_ref[...] = pltpu.repeat(s1_ref[0], _TB // 128, axis=1)

    xr0 = xr_ref[0, 0]
    xi0 = xi_ref[0, 0]
    xr1 = xr_ref[0, 1]
    xi1 = xi_ref[0, 1]

    d2_0 = xr0 * xr0 + xi0 * xi0
    d2_1 = xr1 * xr1 + xi1 * xi1

    liota = jax.lax.broadcasted_iota(jnp.int32, (_F, _TB), 1)
    y = jnp.where(liota < _T - t * _TB, d2_0 * a_ref[...], 0.0)
    d = 1
    for k in range(_NSTEP):
        y = y + dec_ref[k] * jnp.roll(y, d, axis=1)
        d *= 2

    s = y + p_ref[...] * carry_ref[...]
    carry_ref[...] = jnp.broadcast_to(s[:, _TB - 1:_TB], s.shape)

    smooth = jnp.sqrt(s)
    sm_ref[0] = smooth

    wr = pltpu.repeat(w_ref[...], _TB // 128, axis=2)
    br = pltpu.repeat(b_ref[...], _TB // 128, axis=2)
    inv0 = 1.0 / (smooth + 1e-8) * wr[0]
    inv1 = 1.0 / (jnp.sqrt(d2_1) + 1e-8) * wr[1]
    rr_ref[0, 0] = xr0 * inv0 + br[0]
    ri_ref[0, 0] = xi0 * inv0 + br[0]
    rr_ref[0, 1] = xr1 * inv1 + br[1]
    ri_ref[0, 1] = xi1 * inv1 + br[1]


def kernel(input, s_1, weights, bias, alpha_param):
    B, C, T, F, TB = _B, _C, _T, _F, _TB

    x5 = input.transpose(0, 1, 3, 4, 2)
    xr = x5[:, :, :, 0, :]
    xi = x5[:, :, :, 1, :]

    a = jax.nn.sigmoid(alpha_param.reshape(F))
    la = jnp.log1p(-a)
    liota = jnp.arange(TB, dtype=jnp.float32)
    p_d = jnp.exp(la[:, None] * (liota[None, :] + 1.0))
    decs = []
    d = 1
    for _ in range(_NSTEP):
        decs.append(jnp.where(liota[None, :] >= d,
                              jnp.exp(la * float(d))[:, None], 0.0))
        d *= 2
    dec_d = jnp.stack(decs, axis=0)

    a_full = jnp.broadcast_to(a[:, None], (F, TB))
    s1_b = jnp.broadcast_to(s_1.reshape(B, F, 1), (B, F, 128))
    w_b = jnp.broadcast_to(weights.reshape(C, F, 1), (C, F, 128))
    b_b = jnp.broadcast_to(bias.reshape(C, F, 1), (C, F, 128))

    res_r, res_i, smooth = pl.pallas_call(
        _ema_norm_kernel,
        grid=(B, _NT),
        in_specs=[
            pl.BlockSpec((1, C, F, TB), lambda b, t: (b, 0, 0, t)),
            pl.BlockSpec((1, C, F, TB), lambda b, t: (b, 0, 0, t)),
            pl.BlockSpec((1, F, 128), lambda b, t: (b, 0, 0)),
            pl.BlockSpec((F, TB), lambda b, t: (0, 0)),
            pl.BlockSpec((F, TB), lambda b, t: (0, 0)),
            pl.BlockSpec((_NSTEP, F, TB), lambda b, t: (0, 0, 0)),
            pl.BlockSpec((C, F, 128), lambda b, t: (0, 0, 0)),
            pl.BlockSpec((C, F, 128), lambda b, t: (0, 0, 0)),
        ],
        out_specs=[
            pl.BlockSpec((1, C, F, TB), lambda b, t: (b, 0, 0, t)),
            pl.BlockSpec((1, C, F, TB), lambda b, t: (b, 0, 0, t)),
            pl.BlockSpec((1, F, TB), lambda b, t: (b, 0, t)),
        ],
        out_shape=[
            jax.ShapeDtypeStruct((B, C, F, T), jnp.float32),
            jax.ShapeDtypeStruct((B, C, F, T), jnp.float32),
            jax.ShapeDtypeStruct((B, F, T), jnp.float32),
        ],
        scratch_shapes=[pltpu.VMEM((F, TB), jnp.float32)],
        compiler_params=pltpu.CompilerParams(
            dimension_semantics=("parallel", "arbitrary"),
        ),
        name="ema_norm",
    )(xr, xi, s1_b, a_full, p_d, dec_d, w_b, b_b)

    res = jnp.stack([res_r, res_i], axis=3).transpose(0, 1, 4, 2, 3)
    smooth_data = smooth.transpose(0, 2, 1).reshape(B, 1, T, F, 1)
    s_final = (smooth[:, :, T - 1] ** 2).reshape(B, 1, F, 1)
    return res, s_final, smooth_data


# speedup vs baseline: 2.7856x; 1.4104x over previous
import jax
import jax.numpy as jnp
from jax.experimental import pallas as pl
from jax.experimental.pallas import tpu as pltpu

_B, _C, _T, _F = 16, 2, 2000, 257
_TB = 512
_NT = -(-_T // _TB)
_NSTEP = 9


def _ema_norm_kernel(xr_ref, xi_ref, s1_ref, a_ref, p_ref, dec_ref,
                     w_ref, b_ref, res_ref, sm_ref, carry_ref):
    t = pl.program_id(1)

    @pl.when(t == 0)
    def _():
        carry_ref[...] = pltpu.repeat(s1_ref[0], _TB // 128, axis=1)

    xr0 = xr_ref[0, 0, 0]
    xi0 = xi_ref[0, 0, 0]
    xr1 = xr_ref[0, 0, 1]
    xi1 = xi_ref[0, 0, 1]

    d2_0 = xr0 * xr0 + xi0 * xi0
    d2_1 = xr1 * xr1 + xi1 * xi1

    liota = jax.lax.broadcasted_iota(jnp.int32, (_F, _TB), 1)
    y = jnp.where(liota < _T - t * _TB, d2_0 * a_ref[...], 0.0)
    d = 1
    for k in range(_NSTEP):
        y = y + dec_ref[k] * jnp.roll(y, d, axis=1)
        d *= 2

    s = y + p_ref[...] * carry_ref[...]
    carry_ref[...] = jnp.broadcast_to(s[:, _TB - 1:_TB], s.shape)

    smooth = jnp.sqrt(s)
    sm_ref[0] = smooth

    wr = pltpu.repeat(w_ref[...], _TB // 128, axis=2)
    br = pltpu.repeat(b_ref[...], _TB // 128, axis=2)
    inv0 = 1.0 / (smooth + 1e-8) * wr[0]
    inv1 = 1.0 / (jnp.sqrt(d2_1) + 1e-8) * wr[1]
    res_ref[0, 0, 0] = xr0 * inv0 + br[0]
    res_ref[1, 0, 0] = xi0 * inv0 + br[0]
    res_ref[0, 0, 1] = xr1 * inv1 + br[1]
    res_ref[1, 0, 1] = xi1 * inv1 + br[1]


def kernel(input, s_1, weights, bias, alpha_param):
    B, C, T, F, TB = _B, _C, _T, _F, _TB

    xp = input.transpose(4, 0, 1, 3, 2)

    a = jax.nn.sigmoid(alpha_param.reshape(F))
    la = jnp.log1p(-a)
    liota = jnp.arange(TB, dtype=jnp.float32)
    p_d = jnp.exp(la[:, None] * (liota[None, :] + 1.0))
    decs = []
    d = 1
    for _ in range(_NSTEP):
        decs.append(jnp.where(liota[None, :] >= d,
                              jnp.exp(la * float(d))[:, None], 0.0))
        d *= 2
    dec_d = jnp.stack(decs, axis=0)

    a_full = jnp.broadcast_to(a[:, None], (F, TB))
    s1_b = jnp.broadcast_to(s_1.reshape(B, F, 1), (B, F, 128))
    w_b = jnp.broadcast_to(weights.reshape(C, F, 1), (C, F, 128))
    b_b = jnp.broadcast_to(bias.reshape(C, F, 1), (C, F, 128))

    resp, smooth = pl.pallas_call(
        _ema_norm_kernel,
        grid=(B, _NT),
        in_specs=[
            pl.BlockSpec((1, 1, C, F, TB), lambda b, t: (0, b, 0, 0, t)),
            pl.BlockSpec((1, 1, C, F, TB), lambda b, t: (1, b, 0, 0, t)),
            pl.BlockSpec((1, F, 128), lambda b, t: (b, 0, 0)),
            pl.BlockSpec((F, TB), lambda b, t: (0, 0)),
            pl.BlockSpec((F, TB), lambda b, t: (0, 0)),
            pl.BlockSpec((_NSTEP, F, TB), lambda b, t: (0, 0, 0)),
            pl.BlockSpec((C, F, 128), lambda b, t: (0, 0, 0)),
            pl.BlockSpec((C, F, 128), lambda b, t: (0, 0, 0)),
        ],
        out_specs=[
            pl.BlockSpec((2, 1, C, F, TB), lambda b, t: (0, b, 0, 0, t)),
            pl.BlockSpec((1, F, TB), lambda b, t: (b, 0, t)),
        ],
        out_shape=[
            jax.ShapeDtypeStruct((2, B, C, F, T), jnp.float32),
            jax.ShapeDtypeStruct((B, F, T), jnp.float32),
        ],
        scratch_shapes=[pltpu.VMEM((_F, TB), jnp.float32)],
        compiler_params=pltpu.CompilerParams(
            dimension_semantics=("parallel", "arbitrary"),
        ),
        name="ema_norm",
    )(xp, xp, s1_b, a_full, p_d, dec_d, w_b, b_b)

    res = resp.transpose(1, 2, 4, 3, 0)
    smooth_data = smooth.transpose(0, 2, 1).reshape(B, 1, T, F, 1)
    s_final = (smooth[:, :, T - 1] ** 2).reshape(B, 1, F, 1)
    return res, s_final, smooth_data


# speedup vs baseline: 2.8747x; 1.0320x over previous
import jax
import jax.numpy as jnp
from jax.experimental import pallas as pl
from jax.experimental.pallas import tpu as pltpu

_B, _C, _T, _F = 16, 2, 2000, 257
_TB = 1024
_NT = -(-_T // _TB)
_NSTEP = 10


def _ema_norm_kernel(xr_ref, xi_ref, s1_ref, a_ref, p_ref, dec_ref,
                     w_ref, b_ref, res_ref, sm_ref, carry_ref):
    t = pl.program_id(1)

    @pl.when(t == 0)
    def _():
        carry_ref[...] = pltpu.repeat(s1_ref[0], _TB // 128, axis=1)

    xr0 = xr_ref[0, 0, 0]
    xi0 = xi_ref[0, 0, 0]
    xr1 = xr_ref[0, 0, 1]
    xi1 = xi_ref[0, 0, 1]

    d2_0 = xr0 * xr0 + xi0 * xi0
    d2_1 = xr1 * xr1 + xi1 * xi1

    liota = jax.lax.broadcasted_iota(jnp.int32, (_F, _TB), 1)
    y = jnp.where(liota < _T - t * _TB, d2_0 * a_ref[...], 0.0)
    d = 1
    for k in range(_NSTEP):
        y = y + dec_ref[k] * jnp.roll(y, d, axis=1)
        d *= 2

    s = y + p_ref[...] * carry_ref[...]
    carry_ref[...] = jnp.broadcast_to(s[:, _TB - 1:_TB], s.shape)

    smooth = jnp.sqrt(s)
    sm_ref[0] = smooth

    wr = pltpu.repeat(w_ref[...], _TB // 128, axis=2)
    br = pltpu.repeat(b_ref[...], _TB // 128, axis=2)
    inv0 = 1.0 / (smooth + 1e-8) * wr[0]
    inv1 = 1.0 / (jnp.sqrt(d2_1) + 1e-8) * wr[1]
    res_ref[0, 0, 0] = xr0 * inv0 + br[0]
    res_ref[1, 0, 0] = xi0 * inv0 + br[0]
    res_ref[0, 0, 1] = xr1 * inv1 + br[1]
    res_ref[1, 0, 1] = xi1 * inv1 + br[1]


def kernel(input, s_1, weights, bias, alpha_param):
    B, C, T, F, TB = _B, _C, _T, _F, _TB

    xp = input.transpose(4, 0, 1, 3, 2)

    a = jax.nn.sigmoid(alpha_param.reshape(F))
    la = jnp.log1p(-a)
    liota = jnp.arange(TB, dtype=jnp.float32)
    p_d = jnp.exp(la[:, None] * (liota[None, :] + 1.0))
    decs = []
    d = 1
    for _ in range(_NSTEP):
        decs.append(jnp.where(liota[None, :] >= d,
                              jnp.exp(la * float(d))[:, None], 0.0))
        d *= 2
    dec_d = jnp.stack(decs, axis=0)

    a_full = jnp.broadcast_to(a[:, None], (F, TB))
    s1_b = jnp.broadcast_to(s_1.reshape(B, F, 1), (B, F, 128))
    w_b = jnp.broadcast_to(weights.reshape(C, F, 1), (C, F, 128))
    b_b = jnp.broadcast_to(bias.reshape(C, F, 1), (C, F, 128))

    resp, smooth = pl.pallas_call(
        _ema_norm_kernel,
        grid=(B, _NT),
        in_specs=[
            pl.BlockSpec((1, 1, C, F, TB), lambda b, t: (0, b, 0, 0, t)),
            pl.BlockSpec((1, 1, C, F, TB), lambda b, t: (1, b, 0, 0, t)),
            pl.BlockSpec((1, F, 128), lambda b, t: (b, 0, 0)),
            pl.BlockSpec((F, TB), lambda b, t: (0, 0)),
            pl.BlockSpec((F, TB), lambda b, t: (0, 0)),
            pl.BlockSpec((_NSTEP, F, TB), lambda b, t: (0, 0, 0)),
            pl.BlockSpec((C, F, 128), lambda b, t: (0, 0, 0)),
            pl.BlockSpec((C, F, 128), lambda b, t: (0, 0, 0)),
        ],
        out_specs=[
            pl.BlockSpec((2, 1, C, F, TB), lambda b, t: (0, b, 0, 0, t)),
            pl.BlockSpec((1, F, TB), lambda b, t: (b, 0, t)),
        ],
        out_shape=[
            jax.ShapeDtypeStruct((2, B, C, F, T), jnp.float32),
            jax.ShapeDtypeStruct((B, F, T), jnp.float32),
        ],
        scratch_shapes=[pltpu.VMEM((_F, TB), jnp.float32)],
        compiler_params=pltpu.CompilerParams(
            dimension_semantics=("parallel", "arbitrary"),
            vmem_limit_bytes=56 * 1024 * 1024,
        ),
        name="ema_norm",
    )(xp, xp, s1_b, a_full, p_d, dec_d, w_b, b_b)

    res = resp.transpose(1, 2, 4, 3, 0)
    smooth_data = smooth.transpose(0, 2, 1).reshape(B, 1, T, F, 1)
    s_final = (smooth[:, :, T - 1] ** 2).reshape(B, 1, F, 1)
    return res, s_final, smooth_data
